# Initial kernel scaffold; baseline (speedup 1.0000x reference)
import functools

import jax
import jax.numpy as jnp
from jax import lax
from jax.experimental import pallas as pl
from jax.experimental.pallas import tpu as pltpu
from jax.experimental.pallas import tpu_sc as plsc

D = 32
B_TOTAL = 16384 * 26
NC, NS = 2, 16
NW = NC * NS
B_PER_W = B_TOTAL // NW
CHUNK = 1024
N_CHUNK = B_PER_W // CHUNK

_mesh = plsc.VectorSubcoreMesh(core_axis_name="c", subcore_axis_name="s")


@functools.partial(
    pl.kernel,
    mesh=_mesh,
    out_type=jax.ShapeDtypeStruct((B_TOTAL, D), jnp.float32),
    scratch_types=[
        pltpu.VMEM((CHUNK,), jnp.int32),
        pltpu.VMEM((CHUNK, D), jnp.float32),
        pltpu.SemaphoreType.DMA,
    ],
)
def _gather_rows(idx_hbm, w_hbm, out_hbm, idx_v, rows_v, sem):
    wid = lax.axis_index("s") * NC + lax.axis_index("c")
    base = wid * B_PER_W

    def step(i, carry):
        off = base + i * CHUNK
        pltpu.sync_copy(idx_hbm.at[pl.ds(off, CHUNK)], idx_v)
        pltpu.async_copy(w_hbm.at[idx_v], rows_v, sem).wait()
        pltpu.sync_copy(rows_v, out_hbm.at[pl.ds(off, CHUNK)])
        return carry

    lax.fori_loop(0, N_CHUNK, step, 0)


def kernel(x, weight):
    flat_idx = x.reshape(-1)
    out = _gather_rows(flat_idx, weight)
    return out.reshape(x.shape[0], x.shape[1], D)


# speedup vs baseline: 1.5531x; 1.5531x over previous
import functools

import jax
import jax.numpy as jnp
from jax import lax
from jax.experimental import pallas as pl
from jax.experimental.pallas import tpu as pltpu
from jax.experimental.pallas import tpu_sc as plsc

D = 32
B_TOTAL = 16384 * 26
NC, NS = 2, 16
NW = NC * NS
B_PER_W = B_TOTAL // NW
CHUNK = 1024
N_CHUNK = B_PER_W // CHUNK

_mesh = plsc.VectorSubcoreMesh(core_axis_name="c", subcore_axis_name="s")


@functools.partial(
    pl.kernel,
    mesh=_mesh,
    out_type=jax.ShapeDtypeStruct((B_TOTAL, D), jnp.float32),
    scratch_types=[
        pltpu.VMEM((CHUNK,), jnp.int32),
        pltpu.VMEM((CHUNK, D), jnp.float32),
        pltpu.SemaphoreType.DMA,
    ],
    compiler_params=pltpu.CompilerParams(use_tc_tiling_on_sc=False),
)
def _gather_rows(idx_hbm, w_hbm, out_hbm, idx_v, rows_v, sem):
    wid = lax.axis_index("s") * NC + lax.axis_index("c")
    base = wid * B_PER_W

    def step(i, carry):
        off = base + i * CHUNK
        pltpu.sync_copy(idx_hbm.at[pl.ds(off, CHUNK)], idx_v)
        pltpu.async_copy(w_hbm.at[idx_v], rows_v, sem).wait()
        pltpu.sync_copy(rows_v, out_hbm.at[pl.ds(off, CHUNK)])
        return carry

    lax.fori_loop(0, N_CHUNK, step, 0)


def kernel(x, weight):
    flat_idx = x.reshape(-1)
    out = _gather_rows(flat_idx, weight)
    return out.reshape(x.shape[0], x.shape[1], D)


# speedup vs baseline: 1.5672x; 1.0091x over previous
import functools

import jax
import jax.numpy as jnp
from jax import lax
from jax.experimental import pallas as pl
from jax.experimental.pallas import tpu as pltpu
from jax.experimental.pallas import tpu_sc as plsc

D = 32
B_TOTAL = 16384 * 26
NC, NS = 2, 16
NW = NC * NS
B_PER_W = B_TOTAL // NW
N_CHUNK = 8
CHUNK = B_PER_W // N_CHUNK

_mesh = plsc.VectorSubcoreMesh(core_axis_name="c", subcore_axis_name="s")


@functools.partial(
    pl.kernel,
    mesh=_mesh,
    out_type=jax.ShapeDtypeStruct((B_TOTAL, D), jnp.float32),
    scratch_types=[
        pltpu.VMEM((B_PER_W,), jnp.int32),
        pltpu.VMEM((CHUNK, D), jnp.float32),
        pltpu.VMEM((CHUNK, D), jnp.float32),
        pltpu.SemaphoreType.DMA,
        pltpu.SemaphoreType.DMA,
        pltpu.SemaphoreType.DMA,
        pltpu.SemaphoreType.DMA,
    ],
    compiler_params=pltpu.CompilerParams(use_tc_tiling_on_sc=False),
)
def _gather_rows(idx_hbm, w_hbm, out_hbm, idx_v, rows0, rows1,
                 s_g0, s_g1, s_o0, s_o1):
    wid = lax.axis_index("s") * NC + lax.axis_index("c")
    base = wid * B_PER_W

    rows = (rows0, rows1)
    s_g = (s_g0, s_g1)
    s_o = (s_o0, s_o1)

    pltpu.sync_copy(idx_hbm.at[pl.ds(base, B_PER_W)], idx_v)

    def gather(i):
        return pltpu.async_copy(
            w_hbm.at[idx_v.at[pl.ds(i * CHUNK, CHUNK)]],
            rows[i % 2], s_g[i % 2])

    def writeback(i):
        return pltpu.async_copy(
            rows[i % 2], out_hbm.at[pl.ds(base + i * CHUNK, CHUNK)],
            s_o[i % 2])

    h_g = [None] * N_CHUNK
    h_o = [None] * N_CHUNK
    h_g[0] = gather(0)
    for i in range(N_CHUNK):
        h_g[i].wait()
        h_o[i] = writeback(i)
        if i + 1 < N_CHUNK:
            if i >= 1:
                h_o[i - 1].wait()
            h_g[i + 1] = gather(i + 1)
    h_o[N_CHUNK - 1].wait()


def kernel(x, weight):
    flat_idx = x.reshape(-1)
    out = _gather_rows(flat_idx, weight)
    return out.reshape(x.shape[0], x.shape[1], D)


# speedup vs baseline: 1.6689x; 1.0649x over previous
import functools

import jax
import jax.numpy as jnp
from jax import lax
from jax.experimental import pallas as pl
from jax.experimental.pallas import tpu as pltpu
from jax.experimental.pallas import tpu_sc as plsc

D = 32
NF = 26
BATCH = 16384
NC, NS = 2, 16
NW = NC * NS
BW = BATCH // NW

_mesh = plsc.VectorSubcoreMesh(core_axis_name="c", subcore_axis_name="s")


@functools.partial(
    pl.kernel,
    mesh=_mesh,
    out_type=jax.ShapeDtypeStruct((NF, BATCH, D), jnp.float32),
    scratch_types=[
        pltpu.VMEM((NF, BW), jnp.int32),
        pltpu.VMEM((BW, D), jnp.float32),
        pltpu.VMEM((BW, D), jnp.float32),
        pltpu.SemaphoreType.DMA,
        pltpu.SemaphoreType.DMA,
        pltpu.SemaphoreType.DMA,
        pltpu.SemaphoreType.DMA,
    ],
    compiler_params=pltpu.CompilerParams(use_tc_tiling_on_sc=False),
)
def _gather_rows(xT_hbm, w_hbm, out_hbm, idx_v, rows0, rows1,
                 s_g0, s_g1, s_o0, s_o1):
    wid = lax.axis_index("s") * NC + lax.axis_index("c")
    b0 = wid * BW

    rows = (rows0, rows1)
    s_g = (s_g0, s_g1)
    s_o = (s_o0, s_o1)

    pltpu.sync_copy(xT_hbm.at[:, pl.ds(b0, BW)], idx_v)

    def gather(f, b):
        return pltpu.async_copy(w_hbm.at[idx_v.at[f]], rows[b], s_g[b])

    def writeback(f, b):
        return pltpu.async_copy(rows[b], out_hbm.at[f, pl.ds(b0, BW)],
                                s_o[b])

    h_g = [None] * NF
    h_o = [None] * NF
    h_g[0] = gather(0, 0)
    h_g[1] = gather(1, 1)
    for f in range(NF):
        b = f % 2
        h_g[f].wait()
        h_o[f] = writeback(f, b)
        if f + 2 < NF:
            h_o[f].wait()
            h_g[f + 2] = gather(f + 2, b)
    h_o[NF - 2].wait()
    h_o[NF - 1].wait()


def kernel(x, weight):
    out = _gather_rows(x.T, weight)
    return out.transpose(1, 0, 2)
